# Initial kernel scaffold; baseline (speedup 1.0000x reference)
import jax
import jax.numpy as jnp
from jax.experimental import pallas as pl


def kernel(x, edge_index1, edge_index2, W1, a_l1, a_r1, b1, W2, a_l2, a_r2, b2):
    raise NotImplementedError("write your pallas kernel here")


# speedup vs baseline: 12.5095x; 12.5095x over previous
import functools

import jax
import jax.numpy as jnp
from jax import lax
from jax.experimental import pallas as pl
from jax.experimental.pallas import tpu as pltpu
from jax.experimental.pallas import tpu_sc as plsc

_N, _D, _H, _F = 50000, 64, 4, 64
_HF = _H * _F
_NP = 51200
_RPT = _NP // 16
_E = 800000
_EP = 802816
_EPT = _EP // 16
_B = 512
_NBATCH = _EPT // _B
_CW = 16
_NC = _HF // _CW
_BN = 512
_GRID = _NP // _BN

_f32 = jnp.float32
_HIGH = lax.Precision.HIGHEST


def _dot(a, b):
    return lax.dot_general(a, b, (((1,), (0,)), ((), ())),
                           precision=_HIGH, preferred_element_type=_f32)


def _mm1_body(x_ref, w_ref, al_ref, ar_ref, ft_ref, el_ref, er_ref):
    feat = _dot(x_ref[...], w_ref[...])
    ft_ref[...] = feat
    el_ref[...] = _dot(feat, al_ref[...])
    er_ref[...] = _dot(feat, ar_ref[...])


def _mm1(xp, w, al, ar):
    return pl.pallas_call(
        _mm1_body,
        grid=(_GRID,),
        in_specs=[
            pl.BlockSpec((_BN, _D), lambda i: (i, 0)),
            pl.BlockSpec((_D, _HF), lambda i: (0, 0)),
            pl.BlockSpec((_HF, _H), lambda i: (0, 0)),
            pl.BlockSpec((_HF, _H), lambda i: (0, 0)),
        ],
        out_specs=[
            pl.BlockSpec((_BN, _HF), lambda i: (i, 0)),
            pl.BlockSpec((_BN, _H), lambda i: (i, 0)),
            pl.BlockSpec((_BN, _H), lambda i: (i, 0)),
        ],
        out_shape=[
            jax.ShapeDtypeStruct((_NP, _HF), _f32),
            jax.ShapeDtypeStruct((_NP, _H), _f32),
            jax.ShapeDtypeStruct((_NP, _H), _f32),
        ],
    )(xp, w, al, ar)


def _agg_to_h(acc_ref, den_ref, b_ref, act):
    quarters = []
    for q in range(4):
        o = None
        for h in range(4):
            cq = 4 * h + q
            d = den_ref[:, h:h + 1] + 1e-9
            seg = acc_ref[:, cq * _CW:(cq + 1) * _CW] / d \
                + b_ref[:, cq * _CW:(cq + 1) * _CW]
            if act:
                seg = jnp.tanh(seg)
            o = seg if o is None else o + seg
        quarters.append(o * 0.25)
    return quarters


def _mid_body(acc_ref, den_ref, b_ref, w_ref, al_ref, ar_ref,
              ft_ref, el_ref, er_ref):
    quarters = _agg_to_h(acc_ref, den_ref, b_ref, True)
    feat = None
    for q in range(4):
        p = _dot(quarters[q], w_ref[q * 16:(q + 1) * 16, :])
        feat = p if feat is None else feat + p
    ft_ref[...] = feat
    el_ref[...] = _dot(feat, al_ref[...])
    er_ref[...] = _dot(feat, ar_ref[...])


def _mid(acc_n, den_n, bf, w, al, ar):
    return pl.pallas_call(
        _mid_body,
        grid=(_GRID,),
        in_specs=[
            pl.BlockSpec((_BN, _HF), lambda i: (i, 0)),
            pl.BlockSpec((_BN, _H), lambda i: (i, 0)),
            pl.BlockSpec((1, _HF), lambda i: (0, 0)),
            pl.BlockSpec((_F, _HF), lambda i: (0, 0)),
            pl.BlockSpec((_HF, _H), lambda i: (0, 0)),
            pl.BlockSpec((_HF, _H), lambda i: (0, 0)),
        ],
        out_specs=[
            pl.BlockSpec((_BN, _HF), lambda i: (i, 0)),
            pl.BlockSpec((_BN, _H), lambda i: (i, 0)),
            pl.BlockSpec((_BN, _H), lambda i: (i, 0)),
        ],
        out_shape=[
            jax.ShapeDtypeStruct((_NP, _HF), _f32),
            jax.ShapeDtypeStruct((_NP, _H), _f32),
            jax.ShapeDtypeStruct((_NP, _H), _f32),
        ],
    )(acc_n, den_n, bf, w, al, ar)


def _fin_body(acc_ref, den_ref, b_ref, out_ref):
    quarters = _agg_to_h(acc_ref, den_ref, b_ref, False)
    for q in range(4):
        out_ref[:, q * 16:(q + 1) * 16] = quarters[q]


def _fin(acc_n, den_n, bf):
    return pl.pallas_call(
        _fin_body,
        grid=(_GRID,),
        in_specs=[
            pl.BlockSpec((_BN, _HF), lambda i: (i, 0)),
            pl.BlockSpec((_BN, _H), lambda i: (i, 0)),
            pl.BlockSpec((1, _HF), lambda i: (0, 0)),
        ],
        out_specs=pl.BlockSpec((_BN, _F), lambda i: (i, 0)),
        out_shape=jax.ShapeDtypeStruct((_NP, _F), _f32),
    )(acc_n, den_n, bf)


_MESH = plsc.VectorSubcoreMesh(core_axis_name="c", subcore_axis_name="s")
_SCPARAMS = pltpu.CompilerParams(needs_layout_passes=False,
                                 use_tc_tiling_on_sc=False)
_B1 = 512
_NB1 = _EPT // _B1
_B2 = 1024
_KC2 = _B2 // 128
_NB2 = _EPT // _B2


def _sc_ex_body(el_h, er_h, src_h, dst_h, zden_h, ex_o, den_o,
                el_v, er_v, srcb, dstb, exw, den_s, sem):
    del sem
    c = lax.axis_index("c")
    s = lax.axis_index("s")
    row0 = s * _RPT
    ebase0 = s * _EPT

    for hl in range(2):
        head = c * 2 + hl
        pltpu.sync_copy(el_h.at[head], el_v)
        pltpu.sync_copy(er_h.at[head], er_v)
        pltpu.sync_copy(zden_h, den_s.at[pl.ds(row0, _RPT)])
        plsc.subcore_barrier()

        ex_oh = ex_o.at[head]

        def batch(bi, _):
            base = ebase0 + bi * _B1
            for k in range(_B1 // 128):
                pltpu.sync_copy(src_h.at[pl.ds(base + k * 128, 128)],
                                srcb.at[k])
                pltpu.sync_copy(dst_h.at[pl.ds(base + k * 128, 128)],
                                dstb.at[k])
            for k in range(_B1 // 128):
                @plsc.parallel_loop(0, 8)
                def _ex(i, k=k):
                    sv = srcb[k, pl.ds(i * 16, 16)]
                    dv = dstb[k, pl.ds(i * 16, 16)]
                    e = plsc.load_gather(el_v, [sv]) \
                        + plsc.load_gather(er_v, [dv])
                    e = jnp.where(e >= 0.0, e, 0.2 * e)
                    exw[k, pl.ds(i * 16, 16)] = jnp.exp(e)
            for k in range(_B1 // 128):
                pltpu.sync_copy(exw.at[k],
                                ex_oh.at[pl.ds(base + k * 128, 128)])
                pltpu.sync_copy(exw.at[k], den_s.at[dstb.at[k]], add=True)
            return 0

        lax.fori_loop(0, _NB1, batch, 0)
        plsc.subcore_barrier()
        pltpu.sync_copy(den_s.at[pl.ds(row0, _RPT)],
                        den_o.at[head].at[pl.ds(row0, _RPT)])


_sc_ex = functools.partial(
    pl.kernel,
    out_type=[jax.ShapeDtypeStruct((4, _EP), _f32),
              jax.ShapeDtypeStruct((4, _NP), _f32)],
    mesh=_MESH,
    compiler_params=_SCPARAMS,
    scratch_types=[
        pltpu.VMEM((_NP,), _f32),
        pltpu.VMEM((_NP,), _f32),
        pltpu.VMEM((4, 128), jnp.int32),
        pltpu.VMEM((4, 128), jnp.int32),
        pltpu.VMEM((4, 128), _f32),
        pltpu.VMEM_SHARED((_NP,), _f32),
        pltpu.SemaphoreType.DMA,
    ],
)(_sc_ex_body)


def _sc_agg_body(feat_h, ex_h, src_h, dst_h, zacc_h, acc_o,
                 srcb, dstb, wbuf, exv, acc_s, sem):
    c = lax.axis_index("c")
    s = lax.axis_index("s")
    row0 = s * _RPT
    ebase0 = s * _EPT

    for cl in range(8):
        combo = c * 8 + cl
        head = c * 2 + cl // 4
        pltpu.sync_copy(zacc_h, acc_s.at[pl.ds(row0, _RPT)])
        plsc.subcore_barrier()

        feat_c = feat_h.at[combo]
        ex_head = ex_h.at[head]

        def batch(bi, _, feat_c=feat_c, ex_head=ex_head):
            base = ebase0 + bi * _B2
            for k in range(_KC2):
                pltpu.sync_copy(src_h.at[pl.ds(base + k * 128, 128)],
                                srcb.at[k])
                pltpu.sync_copy(dst_h.at[pl.ds(base + k * 128, 128)],
                                dstb.at[k])
                pltpu.sync_copy(ex_head.at[pl.ds(base + k * 128, 128)],
                                exv.at[k])
            cps = []
            for k in range(_KC2):
                cps.append(pltpu.async_copy(
                    feat_c.at[srcb.at[k]],
                    wbuf.at[pl.ds(k * 128, 128)], sem))
            for cp in cps:
                cp.wait()
            for k in range(_KC2):
                @plsc.parallel_loop(0, 128, unroll=8)
                def _scale(r, k=k):
                    j = k * 128 + r
                    ki = jnp.full((16,), k, jnp.int32)
                    ri = jnp.full((16,), r, jnp.int32)
                    exs = plsc.load_gather(exv, [ki, ri])
                    wbuf[j, 0:16] = wbuf[j, 0:16] * exs
            for k in range(_KC2):
                pltpu.sync_copy(wbuf.at[pl.ds(k * 128, 128)],
                                acc_s.at[dstb.at[k]], add=True)
            return 0

        lax.fori_loop(0, _NB2, batch, 0)
        plsc.subcore_barrier()

        pltpu.sync_copy(acc_s.at[pl.ds(row0, _RPT)],
                        acc_o.at[combo].at[pl.ds(row0, _RPT)])


_sc_agg = functools.partial(
    pl.kernel,
    out_type=jax.ShapeDtypeStruct((_NC, _NP, _CW), _f32),
    mesh=_MESH,
    compiler_params=_SCPARAMS,
    scratch_types=[
        pltpu.VMEM((_KC2, 128), jnp.int32),
        pltpu.VMEM((_KC2, 128), jnp.int32),
        pltpu.VMEM((_B2, _CW), _f32),
        pltpu.VMEM((_KC2, 128), _f32),
        pltpu.VMEM_SHARED((_NP, _CW), _f32),
        pltpu.SemaphoreType.DMA,
    ],
)(_sc_agg_body)


def _sc_layer(feat_t, el_t, er_t, src, dst, zacc, zden):
    ex, den = _sc_ex(el_t, er_t, src, dst, zden)
    acc = _sc_agg(feat_t, ex, src, dst, zacc)
    return acc, den


def _prep_edges(ei):
    pad = jnp.full((_EP - _E,), _N, jnp.int32)
    src = jnp.concatenate([ei[0].astype(jnp.int32), pad])
    dst = jnp.concatenate([ei[1].astype(jnp.int32), pad])
    return src, dst


def _amat(a):
    return (a[:, :, None] * jnp.eye(_H, dtype=_f32)[:, None, :]).reshape(_HF, _H)


def _to_tables(feat):
    return jnp.transpose(feat.reshape(_NP, _NC, _CW), (1, 0, 2))


def _from_tables(acc):
    return jnp.transpose(acc, (1, 0, 2)).reshape(_NP, _HF)


@jax.jit
def kernel(x, edge_index1, edge_index2, W1, a_l1, a_r1, b1,
           W2, a_l2, a_r2, b2):
    xp = jnp.pad(x, ((0, _NP - _N), (0, 0)))
    src1, dst1 = _prep_edges(edge_index1)
    src2, dst2 = _prep_edges(edge_index2)

    ft1, el1, er1 = _mm1(xp, W1, _amat(a_l1), _amat(a_r1))
    zacc = jnp.zeros((_RPT, _CW), _f32)
    zden = jnp.zeros((_RPT,), _f32)
    acc1, den1 = _sc_layer(_to_tables(ft1), el1.T, er1.T, src1, dst1,
                           zacc, zden)
    ft2, el2, er2 = _mid(_from_tables(acc1), den1.T, b1.reshape(1, _HF),
                         W2, _amat(a_l2), _amat(a_r2))
    acc2, den2 = _sc_layer(_to_tables(ft2), el2.T, er2.T, src2, dst2,
                           zacc, zden)
    out = _fin(_from_tables(acc2), den2.T, b2.reshape(1, _HF))
    return out[:_N]


# speedup vs baseline: 26.1878x; 2.0934x over previous
import functools

import jax
import jax.numpy as jnp
from jax import lax
from jax.experimental import pallas as pl
from jax.experimental.pallas import tpu as pltpu
from jax.experimental.pallas import tpu_sc as plsc

_N, _D, _H, _F = 50000, 64, 4, 64
_HF = _H * _F
_NP = 51200
_RPT = _NP // 16
_E = 800000
_EP = 802816
_EPT = _EP // 16
_B = 512
_NBATCH = _EPT // _B
_CW = 16
_NC = _HF // _CW
_BN = 512
_GRID = _NP // _BN

_f32 = jnp.float32
_HIGH = lax.Precision.HIGHEST


def _dot(a, b):
    return lax.dot_general(a, b, (((1,), (0,)), ((), ())),
                           precision=_HIGH, preferred_element_type=_f32)


def _mm1_body(x_ref, w_ref, al_ref, ar_ref, ft_ref, el_ref, er_ref):
    feat = _dot(x_ref[...], w_ref[...])
    ft_ref[...] = feat
    el_ref[...] = _dot(feat, al_ref[...])
    er_ref[...] = _dot(feat, ar_ref[...])


def _mm1(xp, w, al, ar):
    return pl.pallas_call(
        _mm1_body,
        grid=(_GRID,),
        in_specs=[
            pl.BlockSpec((_BN, _D), lambda i: (i, 0)),
            pl.BlockSpec((_D, _HF), lambda i: (0, 0)),
            pl.BlockSpec((_HF, _H), lambda i: (0, 0)),
            pl.BlockSpec((_HF, _H), lambda i: (0, 0)),
        ],
        out_specs=[
            pl.BlockSpec((_BN, _HF), lambda i: (i, 0)),
            pl.BlockSpec((_BN, _H), lambda i: (i, 0)),
            pl.BlockSpec((_BN, _H), lambda i: (i, 0)),
        ],
        out_shape=[
            jax.ShapeDtypeStruct((_NP, _HF), _f32),
            jax.ShapeDtypeStruct((_NP, _H), _f32),
            jax.ShapeDtypeStruct((_NP, _H), _f32),
        ],
    )(xp, w, al, ar)


def _agg_to_h(acc_ref, den_ref, b_ref, act):
    quarters = []
    for q in range(4):
        o = None
        for h in range(4):
            cq = 4 * h + q
            d = den_ref[:, h:h + 1] + 1e-9
            seg = acc_ref[:, cq * _CW:(cq + 1) * _CW] / d \
                + b_ref[:, cq * _CW:(cq + 1) * _CW]
            if act:
                seg = jnp.tanh(seg)
            o = seg if o is None else o + seg
        quarters.append(o * 0.25)
    return quarters


def _mid_body(acc_ref, den_ref, b_ref, w_ref, al_ref, ar_ref,
              ft_ref, el_ref, er_ref):
    quarters = _agg_to_h(acc_ref, den_ref, b_ref, True)
    feat = None
    for q in range(4):
        p = _dot(quarters[q], w_ref[q * 16:(q + 1) * 16, :])
        feat = p if feat is None else feat + p
    ft_ref[...] = feat
    el_ref[...] = _dot(feat, al_ref[...])
    er_ref[...] = _dot(feat, ar_ref[...])


def _mid(acc_n, den_n, bf, w, al, ar):
    return pl.pallas_call(
        _mid_body,
        grid=(_GRID,),
        in_specs=[
            pl.BlockSpec((_BN, _HF), lambda i: (i, 0)),
            pl.BlockSpec((_BN, _H), lambda i: (i, 0)),
            pl.BlockSpec((1, _HF), lambda i: (0, 0)),
            pl.BlockSpec((_F, _HF), lambda i: (0, 0)),
            pl.BlockSpec((_HF, _H), lambda i: (0, 0)),
            pl.BlockSpec((_HF, _H), lambda i: (0, 0)),
        ],
        out_specs=[
            pl.BlockSpec((_BN, _HF), lambda i: (i, 0)),
            pl.BlockSpec((_BN, _H), lambda i: (i, 0)),
            pl.BlockSpec((_BN, _H), lambda i: (i, 0)),
        ],
        out_shape=[
            jax.ShapeDtypeStruct((_NP, _HF), _f32),
            jax.ShapeDtypeStruct((_NP, _H), _f32),
            jax.ShapeDtypeStruct((_NP, _H), _f32),
        ],
    )(acc_n, den_n, bf, w, al, ar)


def _fin_body(acc_ref, den_ref, b_ref, out_ref):
    quarters = _agg_to_h(acc_ref, den_ref, b_ref, False)
    for q in range(4):
        out_ref[:, q * 16:(q + 1) * 16] = quarters[q]


def _fin(acc_n, den_n, bf):
    return pl.pallas_call(
        _fin_body,
        grid=(_GRID,),
        in_specs=[
            pl.BlockSpec((_BN, _HF), lambda i: (i, 0)),
            pl.BlockSpec((_BN, _H), lambda i: (i, 0)),
            pl.BlockSpec((1, _HF), lambda i: (0, 0)),
        ],
        out_specs=pl.BlockSpec((_BN, _F), lambda i: (i, 0)),
        out_shape=jax.ShapeDtypeStruct((_NP, _F), _f32),
    )(acc_n, den_n, bf)


_MESH = plsc.VectorSubcoreMesh(core_axis_name="c", subcore_axis_name="s")
_SCPARAMS = pltpu.CompilerParams(needs_layout_passes=False,
                                 use_tc_tiling_on_sc=False)
_B1 = 1024
_NB1 = _EPT // _B1
_B2 = 1024
_KC2 = _B2 // 128
_NB2 = _EPT // _B2
_ECH = (_EP + _B2) // 128
_CPT = _EPT // 128


def _sc_ex_body(el_h, er_h, src_h, dst2_h, zden_h, ex_o, den_o,
                el_v, er_v, srcb, dstb, exw, den_s, sem):
    del sem
    c = lax.axis_index("c")
    s = lax.axis_index("s")
    row0 = s * _RPT
    ebase0 = s * _EPT
    cb0 = s * _CPT

    for hl in range(2):
        head = c * 2 + hl
        pltpu.sync_copy(el_h.at[head], el_v)
        pltpu.sync_copy(er_h.at[head], er_v)
        pltpu.sync_copy(zden_h, den_s.at[pl.ds(row0, _RPT)])
        plsc.subcore_barrier()

        ex_oh = ex_o.at[head]

        def batch(bi, _):
            base = ebase0 + bi * _B1
            cb = cb0 + bi * (_B1 // 128)
            pltpu.sync_copy(src_h.at[pl.ds(base, _B1)], srcb)
            pltpu.sync_copy(dst2_h.at[pl.ds(cb, _B1 // 128)], dstb)
            for k in range(_B1 // 128):
                @plsc.parallel_loop(0, 8)
                def _ex(i, k=k):
                    sv = srcb[pl.ds(k * 128 + i * 16, 16)]
                    dv = dstb[k, pl.ds(i * 16, 16)]
                    e = plsc.load_gather(el_v, [sv]) \
                        + plsc.load_gather(er_v, [dv])
                    e = jnp.where(e >= 0.0, e, 0.2 * e)
                    exw[k, pl.ds(i * 16, 16)] = jnp.exp(e)
            pltpu.sync_copy(exw, ex_oh.at[pl.ds(cb, _B1 // 128)])
            for k in range(_B1 // 128):
                pltpu.sync_copy(exw.at[k], den_s.at[dstb.at[k]], add=True)
            return 0

        lax.fori_loop(0, _NB1, batch, 0)
        plsc.subcore_barrier()
        pltpu.sync_copy(den_s.at[pl.ds(row0, _RPT)],
                        den_o.at[head].at[pl.ds(row0, _RPT)])


_sc_ex = functools.partial(
    pl.kernel,
    out_type=[jax.ShapeDtypeStruct((4, _ECH, 128), _f32),
              jax.ShapeDtypeStruct((4, _NP), _f32)],
    mesh=_MESH,
    compiler_params=_SCPARAMS,
    scratch_types=[
        pltpu.VMEM((_NP,), _f32),
        pltpu.VMEM((_NP,), _f32),
        pltpu.VMEM((_B1,), jnp.int32),
        pltpu.VMEM((_B1 // 128, 128), jnp.int32),
        pltpu.VMEM((_B1 // 128, 128), _f32),
        pltpu.VMEM_SHARED((_NP,), _f32),
        pltpu.SemaphoreType.DMA,
    ],
)(_sc_ex_body)


def _sc_agg_body(feat_h, ex_h, src_h, dst2_h, zacc_h, acc_o,
                 srcA, srcB, dstA, dstB, exA, exB, wA, wB, acc_s,
                 semIA, semIB, semGA, semGB):
    c = lax.axis_index("c")
    s = lax.axis_index("s")
    row0 = s * _RPT
    ebase0 = s * _EPT
    cb0 = s * _CPT

    def combo_loop(cl, _):
        combo = c * 8 + cl
        head = c * 2 + cl // 4
        pltpu.sync_copy(zacc_h, acc_s.at[pl.ds(row0, _RPT)])
        plsc.subcore_barrier()

        feat_c = feat_h.at[combo]
        ex_hd = ex_h.at[head]

        def load_idx(bi, srcX, dstX, exX, semI):
            base = ebase0 + bi * _B2
            cb = cb0 + bi * _KC2
            pltpu.async_copy(src_h.at[pl.ds(base, _B2)], srcX, semI)
            pltpu.async_copy(dst2_h.at[pl.ds(cb, _KC2)], dstX, semI)
            pltpu.async_copy(ex_hd.at[pl.ds(cb, _KC2)], exX, semI)

        def wait_idx(srcX, dstX, exX, semI):
            pltpu.make_async_copy(src_h.at[pl.ds(0, _B2)], srcX, semI).wait()
            pltpu.make_async_copy(dst2_h.at[pl.ds(0, _KC2)], dstX,
                                  semI).wait()
            pltpu.make_async_copy(ex_hd.at[pl.ds(0, _KC2)], exX, semI).wait()

        def fire_g(srcX, wX, semG):
            for k in range(_KC2):
                pltpu.async_copy(feat_c.at[srcX.at[pl.ds(k * 128, 128)]],
                                 wX.at[pl.ds(k * 128, 128)], semG)

        def wait_g(srcX, wX, semG):
            for k in range(_KC2):
                pltpu.make_async_copy(
                    feat_c.at[srcX.at[pl.ds(k * 128, 128)]],
                    wX.at[pl.ds(k * 128, 128)], semG).wait()

        def process(dstX, exX, wX):
            for k in range(_KC2):
                @plsc.parallel_loop(0, 8)
                def _scale(i, k=k):
                    exs = exX[k, pl.ds(i * 16, 16)]
                    ridx = (k * 128 + i * 16) + lax.iota(jnp.int32, 16)
                    for col in range(16):
                        cidx = jnp.full((16,), col, jnp.int32)
                        v = plsc.load_gather(wX, [ridx, cidx])
                        plsc.store_scatter(wX, [ridx, cidx], v * exs)
            for k in range(_KC2):
                pltpu.sync_copy(wX.at[pl.ds(k * 128, 128)],
                                acc_s.at[dstX.at[k]], add=True)

        load_idx(0, srcA, dstA, exA, semIA)
        load_idx(1, srcB, dstB, exB, semIB)
        wait_idx(srcA, dstA, exA, semIA)
        fire_g(srcA, wA, semGA)

        def pair(g, _):
            wait_idx(srcB, dstB, exB, semIB)
            fire_g(srcB, wB, semGB)
            wait_g(srcA, wA, semGA)
            process(dstA, exA, wA)
            load_idx(2 * g + 2, srcA, dstA, exA, semIA)
            wait_idx(srcA, dstA, exA, semIA)
            fire_g(srcA, wA, semGA)
            wait_g(srcB, wB, semGB)
            process(dstB, exB, wB)

            @pl.when(2 * g + 3 <= _NB2 - 1)
            def _ld():
                load_idx(2 * g + 3, srcB, dstB, exB, semIB)
            return 0

        lax.fori_loop(0, (_NB2 - 1) // 2, pair, 0)
        wait_g(srcA, wA, semGA)
        process(dstA, exA, wA)

        plsc.subcore_barrier()
        pltpu.sync_copy(acc_s.at[pl.ds(row0, _RPT)],
                        acc_o.at[combo].at[pl.ds(row0, _RPT)])
        return 0

    lax.fori_loop(0, 8, combo_loop, 0)


_sc_agg = functools.partial(
    pl.kernel,
    out_type=jax.ShapeDtypeStruct((_NC, _NP, _CW), _f32),
    mesh=_MESH,
    compiler_params=_SCPARAMS,
    scratch_types=[
        pltpu.VMEM((_B2,), jnp.int32),
        pltpu.VMEM((_B2,), jnp.int32),
        pltpu.VMEM((_KC2, 128), jnp.int32),
        pltpu.VMEM((_KC2, 128), jnp.int32),
        pltpu.VMEM((_KC2, 128), _f32),
        pltpu.VMEM((_KC2, 128), _f32),
        pltpu.VMEM((_B2, _CW), _f32),
        pltpu.VMEM((_B2, _CW), _f32),
        pltpu.VMEM_SHARED((_NP, _CW), _f32),
        pltpu.SemaphoreType.DMA,
        pltpu.SemaphoreType.DMA,
        pltpu.SemaphoreType.DMA,
        pltpu.SemaphoreType.DMA,
    ],
)(_sc_agg_body)


def _sc_layer(feat_t, el_t, er_t, src, dst2, zacc, zden):
    ex, den = _sc_ex(el_t, er_t, src, dst2, zden)
    acc = _sc_agg(feat_t, ex, src, dst2, zacc)
    return acc, den


def _prep_edges(ei):
    pad = jnp.full((_EP + _B2 - _E,), _N, jnp.int32)
    src = jnp.concatenate([ei[0].astype(jnp.int32), pad])
    dst = jnp.concatenate([ei[1].astype(jnp.int32), pad])
    return src, dst.reshape(_ECH, 128)


def _amat(a):
    return (a[:, :, None] * jnp.eye(_H, dtype=_f32)[:, None, :]).reshape(_HF, _H)


def _to_tables(feat):
    return jnp.transpose(feat.reshape(_NP, _NC, _CW), (1, 0, 2))


def _from_tables(acc):
    return jnp.transpose(acc, (1, 0, 2)).reshape(_NP, _HF)


@jax.jit
def kernel(x, edge_index1, edge_index2, W1, a_l1, a_r1, b1,
           W2, a_l2, a_r2, b2):
    xp = jnp.pad(x, ((0, _NP - _N), (0, 0)))
    src1, dst1 = _prep_edges(edge_index1)
    src2, dst2 = _prep_edges(edge_index2)

    ft1, el1, er1 = _mm1(xp, W1, _amat(a_l1), _amat(a_r1))
    zacc = jnp.zeros((_RPT, _CW), _f32)
    zden = jnp.zeros((_RPT,), _f32)
    acc1, den1 = _sc_layer(_to_tables(ft1), el1.T, er1.T, src1, dst1,
                           zacc, zden)
    ft2, el2, er2 = _mid(_from_tables(acc1), den1.T, b1.reshape(1, _HF),
                         W2, _amat(a_l2), _amat(a_r2))
    acc2, den2 = _sc_layer(_to_tables(ft2), el2.T, er2.T, src2, dst2,
                           zacc, zden)
    out = _fin(_from_tables(acc2), den2.T, b2.reshape(1, _HF))
    return out[:_N]
